# Initial kernel scaffold; baseline (speedup 1.0000x reference)
import jax
import jax.numpy as jnp
from jax.experimental import pallas as pl


def kernel(node_features, positions, neighbors, neighbor_mask, atom_mask, params):
    raise NotImplementedError("write your pallas kernel here")


# speedup vs baseline: 5.1556x; 5.1556x over previous
import math

import jax
import jax.numpy as jnp
from jax.experimental import pallas as pl
from jax.experimental.pallas import tpu as pltpu

N_ATOM_BASIS = 256
N_FILTERS = 256
N_GAUSSIANS = 50
N_INTERACTIONS = 3
CUTOFF = 5.0
MAX_Z = 100
B, NA, NBH = 8, 512, 64

TA = 16
ET = TA * NBH
NT = NA // TA
NGP = 64

_WIDTH = CUTOFF / (N_GAUSSIANS - 1)
_COEFF = -0.5 / (_WIDTH * _WIDTH)
_LOG2 = math.log(2.0)


def _ssp(x):
    return jax.nn.softplus(x) - _LOG2


def _prologue_body(pos_ref, nf_ref, nbr_ref, msk_ref, wemb_ref, bemb_ref,
                   x0_ref, r_ref):
    t = pl.program_id(1)

    @pl.when(t == 0)
    def _():
        x0_ref[0] = jnp.dot(nf_ref[0], wemb_ref[...],
                            preferred_element_type=jnp.float32) + bemb_ref[...]

    nbr = nbr_ref[0]
    a_iota = jax.lax.broadcasted_iota(jnp.int32, (ET, NA), 1)
    e_iota = jax.lax.broadcasted_iota(jnp.int32, (ET, NA), 0)
    atom = t * TA + e_iota // NBH
    sign = ((atom == a_iota).astype(jnp.float32)
            - (nbr == a_iota).astype(jnp.float32))
    diff = jnp.dot(sign, pos_ref[0],
                   preferred_element_type=jnp.float32)
    d2 = jnp.sum(diff * diff, axis=1, keepdims=True)
    safe = jnp.where(d2 > 0.0, d2, 1.0)
    r = jnp.where(d2 > 0.0, jnp.sqrt(safe), 0.0)
    r = jnp.where(msk_ref[0] != 0.0, r, 0.0)
    r_ref[0] = r


def _interaction_body(x_ref, r_ref, nbr_ref, msk_ref,
                      fw1_ref, fb1_ref, fw2_ref, fb2_ref, in2f_ref,
                      f2w_ref, f2b_ref, dw_ref, db_ref,
                      out_ref, y_ref):
    t = pl.program_id(1)

    @pl.when(t == 0)
    def _():
        y_ref[...] = jnp.dot(x_ref[0], in2f_ref[...],
                             preferred_element_type=jnp.float32)

    r = r_ref[0]
    ki = jax.lax.broadcasted_iota(jnp.int32, (ET, NGP), 1)
    offs = jnp.where(ki < N_GAUSSIANS, ki.astype(jnp.float32) * _WIDTH, 1e6)
    f = jnp.exp(_COEFF * (r - offs) ** 2)

    h = _ssp(jnp.dot(f, fw1_ref[...],
                     preferred_element_type=jnp.float32) + fb1_ref[...])
    w = jnp.dot(h, fw2_ref[...],
                preferred_element_type=jnp.float32) + fb2_ref[...]
    cut = 0.5 * (jnp.cos(r * (math.pi / CUTOFF)) + 1.0)
    cut = jnp.where(r < CUTOFF, cut, 0.0)
    w = w * (cut * msk_ref[0])

    nbr = nbr_ref[0]
    onehot = (nbr == jax.lax.broadcasted_iota(jnp.int32, (ET, NA), 1)
              ).astype(jnp.float32)
    yj = jnp.dot(onehot, y_ref[...],
                 preferred_element_type=jnp.float32)

    seg = (jax.lax.broadcasted_iota(jnp.int32, (TA, ET), 1) // NBH
           == jax.lax.broadcasted_iota(jnp.int32, (TA, ET), 0)
           ).astype(jnp.float32)
    z = jnp.dot(seg, yj * w,
                preferred_element_type=jnp.float32)

    v = _ssp(jnp.dot(z, f2w_ref[...],
                     preferred_element_type=jnp.float32) + f2b_ref[...])
    v = jnp.dot(v, dw_ref[...],
                preferred_element_type=jnp.float32) + db_ref[...]
    out_ref[0] = x_ref[0, pl.ds(t * TA, TA), :] + v


def _edge_spec():
    return pl.BlockSpec((1, ET, 1), lambda b, t: (b * NT + t, 0, 0))


def _const_spec(shape):
    nd = len(shape)
    return pl.BlockSpec(shape, lambda b, t: (0,) * nd)


@jax.jit
def kernel(node_features, positions, neighbors, neighbor_mask, atom_mask,
           params):
    del atom_mask
    nbr = neighbors.astype(jnp.int32).reshape(B * NT, ET, 1)
    msk = neighbor_mask.reshape(B * NT, ET, 1)

    grid = (B, NT)
    x0, r = pl.pallas_call(
        _prologue_body,
        grid=grid,
        in_specs=[
            pl.BlockSpec((1, NA, 3), lambda b, t: (b, 0, 0)),
            pl.BlockSpec((1, NA, MAX_Z), lambda b, t: (b, 0, 0)),
            _edge_spec(),
            _edge_spec(),
            _const_spec((MAX_Z, N_ATOM_BASIS)),
            _const_spec((1, N_ATOM_BASIS)),
        ],
        out_specs=[
            pl.BlockSpec((1, NA, N_ATOM_BASIS), lambda b, t: (b, 0, 0)),
            _edge_spec(),
        ],
        out_shape=[
            jax.ShapeDtypeStruct((B, NA, N_ATOM_BASIS), jnp.float32),
            jax.ShapeDtypeStruct((B * NT, ET, 1), jnp.float32),
        ],
    )(positions, node_features, nbr, msk,
      params['W_emb'], params['b_emb'].reshape(1, N_ATOM_BASIS))

    x = x0
    for l in range(N_INTERACTIONS):
        fw1 = jnp.zeros((NGP, N_FILTERS), jnp.float32
                        ).at[:N_GAUSSIANS].set(params['fW1_%d' % l])
        x = pl.pallas_call(
            _interaction_body,
            grid=grid,
            in_specs=[
                pl.BlockSpec((1, NA, N_ATOM_BASIS), lambda b, t: (b, 0, 0)),
                _edge_spec(),
                _edge_spec(),
                _edge_spec(),
                _const_spec((NGP, N_FILTERS)),
                _const_spec((1, N_FILTERS)),
                _const_spec((N_FILTERS, N_FILTERS)),
                _const_spec((1, N_FILTERS)),
                _const_spec((N_ATOM_BASIS, N_FILTERS)),
                _const_spec((N_FILTERS, N_ATOM_BASIS)),
                _const_spec((1, N_ATOM_BASIS)),
                _const_spec((N_ATOM_BASIS, N_ATOM_BASIS)),
                _const_spec((1, N_ATOM_BASIS)),
            ],
            out_specs=pl.BlockSpec((1, TA, N_ATOM_BASIS),
                                   lambda b, t: (b, t, 0)),
            out_shape=jax.ShapeDtypeStruct((B, NA, N_ATOM_BASIS),
                                           jnp.float32),
            scratch_shapes=[pltpu.VMEM((NA, N_FILTERS), jnp.float32)],
        )(x, r, nbr, msk,
          fw1, params['fb1_%d' % l].reshape(1, N_FILTERS),
          params['fW2_%d' % l], params['fb2_%d' % l].reshape(1, N_FILTERS),
          params['in2f_%d' % l],
          params['f2out_W_%d' % l],
          params['f2out_b_%d' % l].reshape(1, N_ATOM_BASIS),
          params['dense_W_%d' % l],
          params['dense_b_%d' % l].reshape(1, N_ATOM_BASIS))
    return x


# speedup vs baseline: 7.1741x; 1.3915x over previous
import math

import jax
import jax.numpy as jnp
from jax.experimental import pallas as pl
from jax.experimental.pallas import tpu as pltpu

N_ATOM_BASIS = 256
N_FILTERS = 256
N_GAUSSIANS = 50
N_INTERACTIONS = 3
CUTOFF = 5.0
MAX_Z = 100
B, NA, NBH = 8, 512, 64

TA = 16
ET = TA * NBH
NT = NA // TA
NGP = 64

_WIDTH = CUTOFF / (N_GAUSSIANS - 1)
_COEFF = -0.5 / (_WIDTH * _WIDTH)
_LOG2 = math.log(2.0)


def _ssp(x):
    return jax.nn.softplus(x) - _LOG2


def _prologue_body(pos_ref, nf_ref, nbr_ref, msk_ref, wemb_ref, bemb_ref,
                   iota_ref, oloc_ref, offs_ref,
                   x0_ref, f_ref, cm_ref):
    t = pl.program_id(1)

    @pl.when(t == 0)
    def _():
        x0_ref[0] = jnp.dot(nf_ref[0], wemb_ref[...],
                            preferred_element_type=jnp.float32) + bemb_ref[...]

    nbr = nbr_ref[0]
    onehot_j = (nbr == iota_ref[...]).astype(jnp.float32)
    p_j = jnp.dot(onehot_j, pos_ref[0],
                  preferred_element_type=jnp.float32)
    p_i = jnp.dot(oloc_ref[...], pos_ref[0, pl.ds(t * TA, TA), :],
                  preferred_element_type=jnp.float32)
    diff = p_i - p_j
    d2 = jnp.sum(diff * diff, axis=1, keepdims=True)
    safe = jnp.where(d2 > 0.0, d2, 1.0)
    r = jnp.where(d2 > 0.0, jnp.sqrt(safe), 0.0)
    r = jnp.where(msk_ref[0] != 0.0, r, 0.0)

    cut = 0.5 * (jnp.cos(r * (math.pi / CUTOFF)) + 1.0)
    cut = jnp.where(r < CUTOFF, cut, 0.0)
    cm_ref[0] = cut * msk_ref[0]
    f_ref[0] = jnp.exp(_COEFF * (r - offs_ref[...]) ** 2)


def _interaction_body(x_ref, f_ref, cm_ref, nbr_ref, iota_ref, seg_ref,
                      fw1_ref, fb1_ref, fw2_ref, fb2_ref, in2f_ref,
                      f2w_ref, f2b_ref, dw_ref, db_ref,
                      out_ref, y_ref):
    t = pl.program_id(1)

    @pl.when(t == 0)
    def _():
        y_ref[...] = jnp.dot(x_ref[0], in2f_ref[...],
                             preferred_element_type=jnp.float32)

    h = _ssp(jnp.dot(f_ref[0], fw1_ref[...],
                     preferred_element_type=jnp.float32) + fb1_ref[...])
    w = jnp.dot(h, fw2_ref[...],
                preferred_element_type=jnp.float32) + fb2_ref[...]
    w = w * cm_ref[0]

    onehot = (nbr_ref[0] == iota_ref[...]).astype(jnp.float32)
    yj = jnp.dot(onehot, y_ref[...],
                 preferred_element_type=jnp.float32)

    z = jnp.dot(seg_ref[...], yj * w,
                preferred_element_type=jnp.float32)

    v = _ssp(jnp.dot(z, f2w_ref[...],
                     preferred_element_type=jnp.float32) + f2b_ref[...])
    v = jnp.dot(v, dw_ref[...],
                preferred_element_type=jnp.float32) + db_ref[...]
    out_ref[0] = x_ref[0, pl.ds(t * TA, TA), :] + v


def _edge_spec(n=1):
    return pl.BlockSpec((1, ET, n), lambda b, t: (b * NT + t, 0, 0))


def _const_spec(shape):
    nd = len(shape)
    return pl.BlockSpec(shape, lambda b, t: (0,) * nd)


@jax.jit
def kernel(node_features, positions, neighbors, neighbor_mask, atom_mask,
           params):
    del atom_mask
    nbr = neighbors.astype(jnp.int32).reshape(B * NT, ET, 1)
    msk = neighbor_mask.reshape(B * NT, ET, 1)

    iota_row = jnp.arange(NA, dtype=jnp.int32).reshape(1, NA)
    e_atom = jnp.arange(ET, dtype=jnp.int32) // NBH
    oloc = jax.nn.one_hot(e_atom, TA, dtype=jnp.float32)
    seg = jax.nn.one_hot(e_atom, TA, dtype=jnp.float32).T
    offs = jnp.where(jnp.arange(NGP) < N_GAUSSIANS,
                     jnp.arange(NGP, dtype=jnp.float32) * _WIDTH,
                     1e6).astype(jnp.float32).reshape(1, NGP)

    grid = (B, NT)
    x0, f, cm = pl.pallas_call(
        _prologue_body,
        grid=grid,
        in_specs=[
            pl.BlockSpec((1, NA, 3), lambda b, t: (b, 0, 0)),
            pl.BlockSpec((1, NA, MAX_Z), lambda b, t: (b, 0, 0)),
            _edge_spec(),
            _edge_spec(),
            _const_spec((MAX_Z, N_ATOM_BASIS)),
            _const_spec((1, N_ATOM_BASIS)),
            _const_spec((1, NA)),
            _const_spec((ET, TA)),
            _const_spec((1, NGP)),
        ],
        out_specs=[
            pl.BlockSpec((1, NA, N_ATOM_BASIS), lambda b, t: (b, 0, 0)),
            _edge_spec(NGP),
            _edge_spec(),
        ],
        out_shape=[
            jax.ShapeDtypeStruct((B, NA, N_ATOM_BASIS), jnp.float32),
            jax.ShapeDtypeStruct((B * NT, ET, NGP), jnp.float32),
            jax.ShapeDtypeStruct((B * NT, ET, 1), jnp.float32),
        ],
    )(positions, node_features, nbr, msk,
      params['W_emb'], params['b_emb'].reshape(1, N_ATOM_BASIS),
      iota_row, oloc, offs)

    x = x0
    for l in range(N_INTERACTIONS):
        fw1 = jnp.zeros((NGP, N_FILTERS), jnp.float32
                        ).at[:N_GAUSSIANS].set(params['fW1_%d' % l])
        x = pl.pallas_call(
            _interaction_body,
            grid=grid,
            in_specs=[
                pl.BlockSpec((1, NA, N_ATOM_BASIS), lambda b, t: (b, 0, 0)),
                _edge_spec(NGP),
                _edge_spec(),
                _edge_spec(),
                _const_spec((1, NA)),
                _const_spec((TA, ET)),
                _const_spec((NGP, N_FILTERS)),
                _const_spec((1, N_FILTERS)),
                _const_spec((N_FILTERS, N_FILTERS)),
                _const_spec((1, N_FILTERS)),
                _const_spec((N_ATOM_BASIS, N_FILTERS)),
                _const_spec((N_FILTERS, N_ATOM_BASIS)),
                _const_spec((1, N_ATOM_BASIS)),
                _const_spec((N_ATOM_BASIS, N_ATOM_BASIS)),
                _const_spec((1, N_ATOM_BASIS)),
            ],
            out_specs=pl.BlockSpec((1, TA, N_ATOM_BASIS),
                                   lambda b, t: (b, t, 0)),
            out_shape=jax.ShapeDtypeStruct((B, NA, N_ATOM_BASIS),
                                           jnp.float32),
            scratch_shapes=[pltpu.VMEM((NA, N_FILTERS), jnp.float32)],
        )(x, f, cm, nbr, iota_row, seg,
          fw1, params['fb1_%d' % l].reshape(1, N_FILTERS),
          params['fW2_%d' % l], params['fb2_%d' % l].reshape(1, N_FILTERS),
          params['in2f_%d' % l],
          params['f2out_W_%d' % l],
          params['f2out_b_%d' % l].reshape(1, N_ATOM_BASIS),
          params['dense_W_%d' % l],
          params['dense_b_%d' % l].reshape(1, N_ATOM_BASIS))
    return x


# speedup vs baseline: 7.3489x; 1.0244x over previous
import math

import jax
import jax.numpy as jnp
from jax.experimental import pallas as pl
from jax.experimental.pallas import tpu as pltpu

N_ATOM_BASIS = 256
N_FILTERS = 256
N_GAUSSIANS = 50
N_INTERACTIONS = 3
CUTOFF = 5.0
MAX_Z = 100
B, NA, NBH = 8, 512, 64

TA = 32
ET = TA * NBH
NT = NA // TA
NGP = 64

_WIDTH = CUTOFF / (N_GAUSSIANS - 1)
_COEFF = -0.5 / (_WIDTH * _WIDTH)
_LOG2 = math.log(2.0)


def _ssp(x):
    return jax.nn.softplus(x) - _LOG2


def _prologue_body(pos_ref, nf_ref, nbr_ref, msk_ref, wemb_ref, bemb_ref,
                   iota_ref, oloc_ref, offs_ref,
                   x0_ref, f_ref, cm_ref):
    t = pl.program_id(1)

    @pl.when(t == 0)
    def _():
        x0_ref[0] = jnp.dot(nf_ref[0], wemb_ref[...],
                            preferred_element_type=jnp.float32) + bemb_ref[...]

    nbr = nbr_ref[0]
    onehot_j = (nbr == iota_ref[...]).astype(jnp.float32)
    p_j = jnp.dot(onehot_j, pos_ref[0],
                  preferred_element_type=jnp.float32)
    p_i = jnp.dot(oloc_ref[...], pos_ref[0, pl.ds(t * TA, TA), :],
                  preferred_element_type=jnp.float32)
    diff = p_i - p_j
    d2 = jnp.sum(diff * diff, axis=1, keepdims=True)
    safe = jnp.where(d2 > 0.0, d2, 1.0)
    r = jnp.where(d2 > 0.0, jnp.sqrt(safe), 0.0)
    r = jnp.where(msk_ref[0] != 0.0, r, 0.0)

    cut = 0.5 * (jnp.cos(r * (math.pi / CUTOFF)) + 1.0)
    cut = jnp.where(r < CUTOFF, cut, 0.0)
    cm_ref[0] = cut * msk_ref[0]
    rb = jnp.dot(r, jnp.ones((1, NGP), jnp.float32),
                 preferred_element_type=jnp.float32)
    f_ref[0] = jnp.exp(_COEFF * (rb - offs_ref[...]) ** 2)


def _interaction_body(x_ref, f_ref, cm_ref, nbr_ref, iota_ref, seg_ref,
                      fw1_ref, fb1_ref, fw2_ref, fb2_ref, in2f_ref,
                      f2w_ref, f2b_ref, dw_ref, db_ref,
                      out_ref, y_ref):
    t = pl.program_id(1)

    @pl.when(t == 0)
    def _():
        y_ref[...] = jnp.dot(x_ref[0], in2f_ref[...],
                             preferred_element_type=jnp.float32)

    h = _ssp(jnp.dot(f_ref[0], fw1_ref[...],
                     preferred_element_type=jnp.float32) + fb1_ref[...])
    w = jnp.dot(h, fw2_ref[...],
                preferred_element_type=jnp.float32) + fb2_ref[...]
    cmb = jnp.dot(cm_ref[0], jnp.ones((1, N_FILTERS), jnp.float32),
                  preferred_element_type=jnp.float32)
    w = w * cmb

    onehot = (nbr_ref[0] == iota_ref[...]).astype(jnp.float32)
    yj = jnp.dot(onehot, y_ref[...],
                 preferred_element_type=jnp.float32)

    z = jnp.dot(seg_ref[...], yj * w,
                preferred_element_type=jnp.float32)

    v = _ssp(jnp.dot(z, f2w_ref[...],
                     preferred_element_type=jnp.float32) + f2b_ref[...])
    v = jnp.dot(v, dw_ref[...],
                preferred_element_type=jnp.float32) + db_ref[...]
    out_ref[0] = x_ref[0, pl.ds(t * TA, TA), :] + v


def _edge_spec(n=1):
    return pl.BlockSpec((1, ET, n), lambda b, t: (b * NT + t, 0, 0))


def _const_spec(shape):
    nd = len(shape)
    return pl.BlockSpec(shape, lambda b, t: (0,) * nd)


@jax.jit
def kernel(node_features, positions, neighbors, neighbor_mask, atom_mask,
           params):
    del atom_mask
    nbr = neighbors.astype(jnp.int32).reshape(B * NT, ET, 1)
    msk = neighbor_mask.reshape(B * NT, ET, 1)

    iota_row = jnp.arange(NA, dtype=jnp.int32).reshape(1, NA)
    e_atom = jnp.arange(ET, dtype=jnp.int32) // NBH
    oloc = jax.nn.one_hot(e_atom, TA, dtype=jnp.float32)
    seg = jax.nn.one_hot(e_atom, TA, dtype=jnp.float32).T
    offs = jnp.where(jnp.arange(NGP) < N_GAUSSIANS,
                     jnp.arange(NGP, dtype=jnp.float32) * _WIDTH,
                     1e6).astype(jnp.float32).reshape(1, NGP)

    grid = (B, NT)
    x0, f, cm = pl.pallas_call(
        _prologue_body,
        grid=grid,
        in_specs=[
            pl.BlockSpec((1, NA, 3), lambda b, t: (b, 0, 0)),
            pl.BlockSpec((1, NA, MAX_Z), lambda b, t: (b, 0, 0)),
            _edge_spec(),
            _edge_spec(),
            _const_spec((MAX_Z, N_ATOM_BASIS)),
            _const_spec((1, N_ATOM_BASIS)),
            _const_spec((1, NA)),
            _const_spec((ET, TA)),
            _const_spec((1, NGP)),
        ],
        out_specs=[
            pl.BlockSpec((1, NA, N_ATOM_BASIS), lambda b, t: (b, 0, 0)),
            _edge_spec(NGP),
            _edge_spec(),
        ],
        out_shape=[
            jax.ShapeDtypeStruct((B, NA, N_ATOM_BASIS), jnp.float32),
            jax.ShapeDtypeStruct((B * NT, ET, NGP), jnp.float32),
            jax.ShapeDtypeStruct((B * NT, ET, 1), jnp.float32),
        ],
    )(positions, node_features, nbr, msk,
      params['W_emb'], params['b_emb'].reshape(1, N_ATOM_BASIS),
      iota_row, oloc, offs)

    x = x0
    for l in range(N_INTERACTIONS):
        fw1 = jnp.zeros((NGP, N_FILTERS), jnp.float32
                        ).at[:N_GAUSSIANS].set(params['fW1_%d' % l])
        x = pl.pallas_call(
            _interaction_body,
            grid=grid,
            in_specs=[
                pl.BlockSpec((1, NA, N_ATOM_BASIS), lambda b, t: (b, 0, 0)),
                _edge_spec(NGP),
                _edge_spec(),
                _edge_spec(),
                _const_spec((1, NA)),
                _const_spec((TA, ET)),
                _const_spec((NGP, N_FILTERS)),
                _const_spec((1, N_FILTERS)),
                _const_spec((N_FILTERS, N_FILTERS)),
                _const_spec((1, N_FILTERS)),
                _const_spec((N_ATOM_BASIS, N_FILTERS)),
                _const_spec((N_FILTERS, N_ATOM_BASIS)),
                _const_spec((1, N_ATOM_BASIS)),
                _const_spec((N_ATOM_BASIS, N_ATOM_BASIS)),
                _const_spec((1, N_ATOM_BASIS)),
            ],
            out_specs=pl.BlockSpec((1, TA, N_ATOM_BASIS),
                                   lambda b, t: (b, t, 0)),
            out_shape=jax.ShapeDtypeStruct((B, NA, N_ATOM_BASIS),
                                           jnp.float32),
            scratch_shapes=[pltpu.VMEM((NA, N_FILTERS), jnp.float32)],
        )(x, f, cm, nbr, iota_row, seg,
          fw1, params['fb1_%d' % l].reshape(1, N_FILTERS),
          params['fW2_%d' % l], params['fb2_%d' % l].reshape(1, N_FILTERS),
          params['in2f_%d' % l],
          params['f2out_W_%d' % l],
          params['f2out_b_%d' % l].reshape(1, N_ATOM_BASIS),
          params['dense_W_%d' % l],
          params['dense_b_%d' % l].reshape(1, N_ATOM_BASIS))
    return x


# speedup vs baseline: 9.9678x; 1.3564x over previous
import functools
import math

import jax
import jax.numpy as jnp
from jax import lax
from jax.experimental import pallas as pl
from jax.experimental.pallas import tpu as pltpu
from jax.experimental.pallas import tpu_sc as plsc

N_ATOM_BASIS = 256
N_FILTERS = 256
N_GAUSSIANS = 50
N_INTERACTIONS = 3
CUTOFF = 5.0
MAX_Z = 100
B, NA, NBH = 8, 512, 64

TA = 32
ET = TA * NBH
NT = NA // TA
NGP = 64

E = B * NA * NBH
NW = 32
EPW = E // NW
APW = (B * NA) // NW

_WIDTH = CUTOFF / (N_GAUSSIANS - 1)
_COEFF = -0.5 / (_WIDTH * _WIDTH)
_LOG2 = math.log(2.0)

_COS_COEF = [(-1.0) ** k * math.pi ** (2 * k) / math.factorial(2 * k)
             for k in range(9)]


def _sc_distances(px, py, pz, nbr_g, msk, own):
    mesh = plsc.VectorSubcoreMesh(core_axis_name="c", subcore_axis_name="s")

    @functools.partial(
        pl.kernel, mesh=mesh,
        out_type=[jax.ShapeDtypeStruct((E,), jnp.float32),
                  jax.ShapeDtypeStruct((E,), jnp.float32)],
        scratch_types=[
            pltpu.VMEM((EPW,), jnp.int32),
            pltpu.VMEM((EPW,), jnp.int32),
            pltpu.VMEM((EPW,), jnp.float32),
            pltpu.VMEM((EPW,), jnp.float32),
            pltpu.VMEM((EPW,), jnp.float32),
            pltpu.VMEM((EPW,), jnp.float32),
            pltpu.VMEM((EPW,), jnp.float32),
            pltpu.VMEM((EPW,), jnp.float32),
            pltpu.VMEM((EPW,), jnp.float32),
            pltpu.VMEM((EPW,), jnp.float32),
            pltpu.VMEM((EPW,), jnp.float32),
            pltpu.SemaphoreType.DMA,
        ])
    def body(px_hbm, py_hbm, pz_hbm, nbr_hbm, msk_hbm, own_hbm,
             r_hbm, cm_hbm,
             idx_v, own_v, xj_v, yj_v, zj_v, mk_v, xi_v, yi_v, zi_v,
             r_v, cm_v, sem):
        wid = lax.axis_index("s") * 2 + lax.axis_index("c")
        ebase = wid * EPW

        pltpu.sync_copy(nbr_hbm.at[pl.ds(ebase, EPW)], idx_v)
        pltpu.sync_copy(own_hbm.at[pl.ds(ebase, EPW)], own_v)
        pltpu.sync_copy(msk_hbm.at[pl.ds(ebase, EPW)], mk_v)

        pltpu.async_copy(px_hbm.at[idx_v], xj_v, sem).wait()
        pltpu.async_copy(py_hbm.at[idx_v], yj_v, sem).wait()
        pltpu.async_copy(pz_hbm.at[idx_v], zj_v, sem).wait()
        pltpu.async_copy(px_hbm.at[own_v], xi_v, sem).wait()
        pltpu.async_copy(py_hbm.at[own_v], yi_v, sem).wait()
        pltpu.async_copy(pz_hbm.at[own_v], zi_v, sem).wait()

        def edge_body(i, carry):
            s = pl.ds(i * 16, 16)
            dx = xj_v[s] - xi_v[s]
            dy = yj_v[s] - yi_v[s]
            dz = zj_v[s] - zi_v[s]
            d2 = dx * dx + dy * dy + dz * dz
            seed = lax.bitcast_convert_type(
                0x5F3759DF - lax.shift_right_logical(
                    lax.bitcast_convert_type(d2, jnp.int32), 1),
                jnp.float32)
            for _ in range(3):
                seed = seed * (1.5 - 0.5 * d2 * seed * seed)
            r = d2 * seed
            good = (d2 > 0.0) & (mk_v[s] != 0.0)
            r = jnp.where(good, r, 0.0)
            u = r * (1.0 / CUTOFF)
            t = u * u
            c = jnp.full((16,), _COS_COEF[8], jnp.float32)
            for k in range(7, -1, -1):
                c = c * t + _COS_COEF[k]
            cut = jnp.where(r < CUTOFF, 0.5 * (c + 1.0), 0.0)
            r_v[s] = r
            cm_v[s] = cut * mk_v[s]
            return carry
        lax.fori_loop(0, EPW // 16, edge_body, 0, unroll=4)

        pltpu.sync_copy(r_v, r_hbm.at[pl.ds(ebase, EPW)])
        pltpu.sync_copy(cm_v, cm_hbm.at[pl.ds(ebase, EPW)])

    return body(px, py, pz, nbr_g, msk, own)


def _prologue_body(r_ref, nf_ref, wemb_ref, bemb_ref, offs_ref,
                   ones_col_ref, x0_ref, f_ref):
    t = pl.program_id(1)

    @pl.when(t == 0)
    def _():
        x0_ref[0] = jnp.dot(nf_ref[0], wemb_ref[...],
                            preferred_element_type=jnp.float32) + bemb_ref[...]

    rb = jnp.dot(r_ref[0], jnp.ones((1, NGP), jnp.float32),
                 preferred_element_type=jnp.float32)
    f = jnp.exp(_COEFF * (rb - offs_ref[...]) ** 2)
    f_ref[0] = f + ones_col_ref[...]


def _interaction_body(x_ref, f_ref, cm_ref, nbr_ref, iota_ref, seg_ref,
                      fw1_ref, fw2_ref, fb2_ref, in2f_ref,
                      f2w_ref, f2b_ref, dw_ref, db_ref,
                      out_ref, y_ref):
    t = pl.program_id(1)

    @pl.when(t == 0)
    def _():
        y_ref[...] = jnp.dot(x_ref[0], in2f_ref[...],
                             preferred_element_type=jnp.float32
                             ).astype(jnp.bfloat16)

    h = jax.nn.softplus(jnp.dot(f_ref[0], fw1_ref[...],
                                preferred_element_type=jnp.float32))
    w = jnp.dot(h, fw2_ref[...],
                preferred_element_type=jnp.float32) + fb2_ref[...]

    onehot = (nbr_ref[0] == iota_ref[...]).astype(jnp.bfloat16)
    yj = jnp.dot(onehot, y_ref[...],
                 preferred_element_type=jnp.float32)

    seg_w = seg_ref[...] * cm_ref[0]
    z = jnp.dot(seg_w, yj * w,
                preferred_element_type=jnp.float32)

    v = jax.nn.softplus(jnp.dot(z, f2w_ref[...],
                                preferred_element_type=jnp.float32)
                        + f2b_ref[...]) - _LOG2
    v = jnp.dot(v, dw_ref[...],
                preferred_element_type=jnp.float32) + db_ref[...]
    out_ref[0] = x_ref[0, pl.ds(t * TA, TA), :] + v


def _edge_spec(n=1):
    return pl.BlockSpec((1, ET, n), lambda b, t: (b * NT + t, 0, 0))


def _row_spec():
    return pl.BlockSpec((1, 1, ET), lambda b, t: (b * NT + t, 0, 0))


def _const_spec(shape):
    nd = len(shape)
    return pl.BlockSpec(shape, lambda b, t: (0,) * nd)


@jax.jit
def kernel(node_features, positions, neighbors, neighbor_mask, atom_mask,
           params):
    del atom_mask
    nbr_col = neighbors.astype(jnp.int32).reshape(B * NT, ET, 1)
    nbr_glob = (neighbors.astype(jnp.int32)
                + jnp.arange(B, dtype=jnp.int32)[:, None, None] * NA
                ).reshape(E)
    msk_flat = neighbor_mask.reshape(E)
    px = positions[:, :, 0].reshape(B * NA)
    py = positions[:, :, 1].reshape(B * NA)
    pz = positions[:, :, 2].reshape(B * NA)
    own_idx = jnp.arange(E, dtype=jnp.int32) // NBH

    r_flat, cm_flat = _sc_distances(px, py, pz, nbr_glob, msk_flat, own_idx)
    r_col = r_flat.reshape(B * NT, ET, 1)
    cm_row = cm_flat.reshape(B * NT, 1, ET)

    iota_row = jnp.arange(NA, dtype=jnp.int32).reshape(1, NA)
    e_atom = jnp.arange(ET, dtype=jnp.int32) // NBH
    seg = jax.nn.one_hot(e_atom, TA, dtype=jnp.float32).T
    offs = jnp.where(jnp.arange(NGP) < N_GAUSSIANS,
                     jnp.arange(NGP, dtype=jnp.float32) * _WIDTH,
                     1e6).astype(jnp.float32).reshape(1, NGP)
    ones_col = (jnp.arange(NGP) == NGP - 1
                ).astype(jnp.float32).reshape(1, NGP)

    grid = (B, NT)
    x0, f = pl.pallas_call(
        _prologue_body,
        grid=grid,
        in_specs=[
            _edge_spec(),
            pl.BlockSpec((1, NA, MAX_Z), lambda b, t: (b, 0, 0)),
            _const_spec((MAX_Z, N_ATOM_BASIS)),
            _const_spec((1, N_ATOM_BASIS)),
            _const_spec((1, NGP)),
            _const_spec((1, NGP)),
        ],
        out_specs=[
            pl.BlockSpec((1, NA, N_ATOM_BASIS), lambda b, t: (b, 0, 0)),
            _edge_spec(NGP),
        ],
        out_shape=[
            jax.ShapeDtypeStruct((B, NA, N_ATOM_BASIS), jnp.float32),
            jax.ShapeDtypeStruct((B * NT, ET, NGP), jnp.float32),
        ],
    )(r_col, node_features,
      params['W_emb'], params['b_emb'].reshape(1, N_ATOM_BASIS),
      offs, ones_col)

    x = x0
    for l in range(N_INTERACTIONS):
        fw1 = jnp.zeros((NGP, N_FILTERS), jnp.float32
                        ).at[:N_GAUSSIANS].set(params['fW1_%d' % l]
                        ).at[NGP - 1].set(params['fb1_%d' % l])
        fb2 = (params['fb2_%d' % l]
               - _LOG2 * jnp.sum(params['fW2_%d' % l], axis=0)
               ).reshape(1, N_FILTERS)
        x = pl.pallas_call(
            _interaction_body,
            grid=grid,
            in_specs=[
                pl.BlockSpec((1, NA, N_ATOM_BASIS), lambda b, t: (b, 0, 0)),
                _edge_spec(NGP),
                _row_spec(),
                _edge_spec(),
                _const_spec((1, NA)),
                _const_spec((TA, ET)),
                _const_spec((NGP, N_FILTERS)),
                _const_spec((N_FILTERS, N_FILTERS)),
                _const_spec((1, N_FILTERS)),
                _const_spec((N_ATOM_BASIS, N_FILTERS)),
                _const_spec((N_FILTERS, N_ATOM_BASIS)),
                _const_spec((1, N_ATOM_BASIS)),
                _const_spec((N_ATOM_BASIS, N_ATOM_BASIS)),
                _const_spec((1, N_ATOM_BASIS)),
            ],
            out_specs=pl.BlockSpec((1, TA, N_ATOM_BASIS),
                                   lambda b, t: (b, t, 0)),
            out_shape=jax.ShapeDtypeStruct((B, NA, N_ATOM_BASIS),
                                           jnp.float32),
            scratch_shapes=[pltpu.VMEM((NA, N_FILTERS), jnp.bfloat16)],
        )(x, f, cm_row, nbr_col, iota_row, seg,
          fw1, params['fW2_%d' % l], fb2,
          params['in2f_%d' % l],
          params['f2out_W_%d' % l],
          params['f2out_b_%d' % l].reshape(1, N_ATOM_BASIS),
          params['dense_W_%d' % l],
          params['dense_b_%d' % l].reshape(1, N_ATOM_BASIS))
    return x


# speedup vs baseline: 10.7413x; 1.0776x over previous
import functools
import math

import jax
import jax.numpy as jnp
from jax import lax
from jax.experimental import pallas as pl
from jax.experimental.pallas import tpu as pltpu
from jax.experimental.pallas import tpu_sc as plsc

N_ATOM_BASIS = 256
N_FILTERS = 256
N_GAUSSIANS = 50
N_INTERACTIONS = 3
CUTOFF = 5.0
MAX_Z = 100
B, NA, NBH = 8, 512, 64

TA = 32
ET = TA * NBH
NT = NA // TA
NGP = 64

E = B * NA * NBH
NW = 32
EPW = E // NW
APW = (B * NA) // NW

_WIDTH = CUTOFF / (N_GAUSSIANS - 1)
_COEFF = -0.5 / (_WIDTH * _WIDTH)
_LOG2 = math.log(2.0)

_COS_COEF = [(-1.0) ** k * math.pi ** (2 * k) / math.factorial(2 * k)
             for k in range(9)]


def _sc_distances(px, py, pz, nbr_g, msk, pxi, pyi, pzi):
    mesh = plsc.VectorSubcoreMesh(core_axis_name="c", subcore_axis_name="s")

    @functools.partial(
        pl.kernel, mesh=mesh,
        out_type=[jax.ShapeDtypeStruct((E,), jnp.float32),
                  jax.ShapeDtypeStruct((E,), jnp.float32)],
        scratch_types=[
            pltpu.VMEM((EPW,), jnp.int32),
            pltpu.VMEM((EPW,), jnp.float32),
            pltpu.VMEM((EPW,), jnp.float32),
            pltpu.VMEM((EPW,), jnp.float32),
            pltpu.VMEM((EPW,), jnp.float32),
            pltpu.VMEM((EPW,), jnp.float32),
            pltpu.VMEM((EPW,), jnp.float32),
            pltpu.VMEM((EPW,), jnp.float32),
            pltpu.VMEM((EPW,), jnp.float32),
            pltpu.VMEM((EPW,), jnp.float32),
            pltpu.SemaphoreType.DMA,
            pltpu.SemaphoreType.DMA,
            pltpu.SemaphoreType.DMA,
            pltpu.SemaphoreType.DMA,
        ])
    def body(px_hbm, py_hbm, pz_hbm, nbr_hbm, msk_hbm,
             xi_hbm, yi_hbm, zi_hbm, r_hbm, cm_hbm,
             idx_v, xj_v, yj_v, zj_v, mk_v, xi_v, yi_v, zi_v,
             r_v, cm_v, s0, s1, s2, s3):
        wid = lax.axis_index("s") * 2 + lax.axis_index("c")
        ebase = wid * EPW

        pltpu.sync_copy(nbr_hbm.at[pl.ds(ebase, EPW)], idx_v)
        pltpu.sync_copy(xi_hbm.at[pl.ds(ebase, EPW)], xi_v)
        pltpu.sync_copy(yi_hbm.at[pl.ds(ebase, EPW)], yi_v)
        pltpu.sync_copy(zi_hbm.at[pl.ds(ebase, EPW)], zi_v)
        pltpu.sync_copy(msk_hbm.at[pl.ds(ebase, EPW)], mk_v)

        pltpu.async_copy(px_hbm.at[idx_v], xj_v, s0).wait()
        pltpu.async_copy(py_hbm.at[idx_v], yj_v, s1).wait()
        pltpu.async_copy(pz_hbm.at[idx_v], zj_v, s2).wait()
        del s3

        def edge_body(i, carry):
            s = pl.ds(i * 16, 16)
            dx = xj_v[s] - xi_v[s]
            dy = yj_v[s] - yi_v[s]
            dz = zj_v[s] - zi_v[s]
            d2 = dx * dx + dy * dy + dz * dz
            seed = lax.bitcast_convert_type(
                0x5F3759DF - lax.shift_right_logical(
                    lax.bitcast_convert_type(d2, jnp.int32), 1),
                jnp.float32)
            for _ in range(3):
                seed = seed * (1.5 - 0.5 * d2 * seed * seed)
            r = d2 * seed
            good = (d2 > 0.0) & (mk_v[s] != 0.0)
            r = jnp.where(good, r, 0.0)
            u = r * (1.0 / CUTOFF)
            t = u * u
            c = jnp.full((16,), _COS_COEF[8], jnp.float32)
            for k in range(7, -1, -1):
                c = c * t + _COS_COEF[k]
            cut = jnp.where(r < CUTOFF, 0.5 * (c + 1.0), 0.0)
            r_v[s] = r
            cm_v[s] = cut * mk_v[s]
            return carry
        lax.fori_loop(0, EPW // 16, edge_body, 0, unroll=4)

        pltpu.sync_copy(r_v, r_hbm.at[pl.ds(ebase, EPW)])
        pltpu.sync_copy(cm_v, cm_hbm.at[pl.ds(ebase, EPW)])

    return body(px, py, pz, nbr_g, msk, pxi, pyi, pzi)


def _prologue_body(r_ref, nf_ref, wemb_ref, bemb_ref, offs_ref,
                   ones_col_ref, x0_ref, f_ref):
    t = pl.program_id(1)

    @pl.when(t == 0)
    def _():
        x0_ref[0] = jnp.dot(nf_ref[0], wemb_ref[...],
                            preferred_element_type=jnp.float32) + bemb_ref[...]

    rb = jnp.dot(r_ref[0], jnp.ones((1, NGP), jnp.float32),
                 preferred_element_type=jnp.float32)
    f = jnp.exp(_COEFF * (rb - offs_ref[...]) ** 2)
    f_ref[0] = f + ones_col_ref[...]


def _interaction_body(x_ref, f_ref, cm_ref, nbr_ref, iota_ref, seg_ref,
                      fw1_ref, fw2_ref, fb2_ref, in2f_ref,
                      f2w_ref, f2b_ref, dw_ref, db_ref,
                      out_ref, y_ref):
    t = pl.program_id(1)

    @pl.when(t == 0)
    def _():
        y_ref[...] = jnp.dot(x_ref[0], in2f_ref[...],
                             preferred_element_type=jnp.float32
                             ).astype(jnp.bfloat16)

    h = jax.nn.softplus(jnp.dot(f_ref[0], fw1_ref[...],
                                preferred_element_type=jnp.float32))
    w = jnp.dot(h, fw2_ref[...],
                preferred_element_type=jnp.float32) + fb2_ref[...]

    onehot = (nbr_ref[0] == iota_ref[...]).astype(jnp.bfloat16)
    yj = jnp.dot(onehot, y_ref[...],
                 preferred_element_type=jnp.float32)

    seg_w = seg_ref[...] * cm_ref[0]
    z = jnp.dot(seg_w, yj * w,
                preferred_element_type=jnp.float32)

    v = jax.nn.softplus(jnp.dot(z, f2w_ref[...],
                                preferred_element_type=jnp.float32)
                        + f2b_ref[...]) - _LOG2
    v = jnp.dot(v, dw_ref[...],
                preferred_element_type=jnp.float32) + db_ref[...]
    out_ref[0] = x_ref[0, pl.ds(t * TA, TA), :] + v


def _edge_spec(n=1):
    return pl.BlockSpec((1, ET, n), lambda b, t: (b * NT + t, 0, 0))


def _row_spec():
    return pl.BlockSpec((1, 1, ET), lambda b, t: (b * NT + t, 0, 0))


def _const_spec(shape):
    nd = len(shape)
    return pl.BlockSpec(shape, lambda b, t: (0,) * nd)


@jax.jit
def kernel(node_features, positions, neighbors, neighbor_mask, atom_mask,
           params):
    del atom_mask
    nbr_col = neighbors.astype(jnp.int32).reshape(B * NT, ET, 1)
    nbr_glob = (neighbors.astype(jnp.int32)
                + jnp.arange(B, dtype=jnp.int32)[:, None, None] * NA
                ).reshape(E)
    msk_flat = neighbor_mask.reshape(E)
    px = positions[:, :, 0].reshape(B * NA)
    py = positions[:, :, 1].reshape(B * NA)
    pz = positions[:, :, 2].reshape(B * NA)
    pos_i = jnp.broadcast_to(positions[:, :, None, :],
                             (B, NA, NBH, 3)).reshape(E, 3)
    pxi, pyi, pzi = pos_i[:, 0], pos_i[:, 1], pos_i[:, 2]

    r_flat, cm_flat = _sc_distances(px, py, pz, nbr_glob, msk_flat,
                                    pxi, pyi, pzi)
    r_col = r_flat.reshape(B * NT, ET, 1)
    cm_row = cm_flat.reshape(B * NT, 1, ET)

    iota_row = jnp.arange(NA, dtype=jnp.int32).reshape(1, NA)
    e_atom = jnp.arange(ET, dtype=jnp.int32) // NBH
    seg = jax.nn.one_hot(e_atom, TA, dtype=jnp.float32).T
    offs = jnp.where(jnp.arange(NGP) < N_GAUSSIANS,
                     jnp.arange(NGP, dtype=jnp.float32) * _WIDTH,
                     1e6).astype(jnp.float32).reshape(1, NGP)
    ones_col = (jnp.arange(NGP) == NGP - 1
                ).astype(jnp.float32).reshape(1, NGP)

    grid = (B, NT)
    x0, f = pl.pallas_call(
        _prologue_body,
        grid=grid,
        in_specs=[
            _edge_spec(),
            pl.BlockSpec((1, NA, MAX_Z), lambda b, t: (b, 0, 0)),
            _const_spec((MAX_Z, N_ATOM_BASIS)),
            _const_spec((1, N_ATOM_BASIS)),
            _const_spec((1, NGP)),
            _const_spec((1, NGP)),
        ],
        out_specs=[
            pl.BlockSpec((1, NA, N_ATOM_BASIS), lambda b, t: (b, 0, 0)),
            _edge_spec(NGP),
        ],
        out_shape=[
            jax.ShapeDtypeStruct((B, NA, N_ATOM_BASIS), jnp.float32),
            jax.ShapeDtypeStruct((B * NT, ET, NGP), jnp.float32),
        ],
    )(r_col, node_features,
      params['W_emb'], params['b_emb'].reshape(1, N_ATOM_BASIS),
      offs, ones_col)

    x = x0
    for l in range(N_INTERACTIONS):
        fw1 = jnp.zeros((NGP, N_FILTERS), jnp.float32
                        ).at[:N_GAUSSIANS].set(params['fW1_%d' % l]
                        ).at[NGP - 1].set(params['fb1_%d' % l])
        fb2 = (params['fb2_%d' % l]
               - _LOG2 * jnp.sum(params['fW2_%d' % l], axis=0)
               ).reshape(1, N_FILTERS)
        x = pl.pallas_call(
            _interaction_body,
            grid=grid,
            in_specs=[
                pl.BlockSpec((1, NA, N_ATOM_BASIS), lambda b, t: (b, 0, 0)),
                _edge_spec(NGP),
                _row_spec(),
                _edge_spec(),
                _const_spec((1, NA)),
                _const_spec((TA, ET)),
                _const_spec((NGP, N_FILTERS)),
                _const_spec((N_FILTERS, N_FILTERS)),
                _const_spec((1, N_FILTERS)),
                _const_spec((N_ATOM_BASIS, N_FILTERS)),
                _const_spec((N_FILTERS, N_ATOM_BASIS)),
                _const_spec((1, N_ATOM_BASIS)),
                _const_spec((N_ATOM_BASIS, N_ATOM_BASIS)),
                _const_spec((1, N_ATOM_BASIS)),
            ],
            out_specs=pl.BlockSpec((1, TA, N_ATOM_BASIS),
                                   lambda b, t: (b, t, 0)),
            out_shape=jax.ShapeDtypeStruct((B, NA, N_ATOM_BASIS),
                                           jnp.float32),
            scratch_shapes=[pltpu.VMEM((NA, N_FILTERS), jnp.bfloat16)],
        )(x, f, cm_row, nbr_col, iota_row, seg,
          fw1, params['fW2_%d' % l], fb2,
          params['in2f_%d' % l],
          params['f2out_W_%d' % l],
          params['f2out_b_%d' % l].reshape(1, N_ATOM_BASIS),
          params['dense_W_%d' % l],
          params['dense_b_%d' % l].reshape(1, N_ATOM_BASIS))
    return x
